# baseline (speedup 1.0000x reference)
import jax
import jax.numpy as jnp
from jax.experimental import pallas as pl
from jax.experimental.pallas import tpu as pltpu

_FWD_WEIGHT = 1.0


def _l1_kernel(x_ref, g_ref, m_ref, o_ref):
    d = jnp.abs(x_ref[...] - g_ref[...])
    mf = m_ref[...].astype(jnp.float32)
    r = d.shape[0] // 8
    m3 = mf.reshape(r, 4, 128)
    idx = jax.lax.broadcasted_iota(jnp.int32, (r, 8, 128), 1) // 2
    mex = jnp.take_along_axis(m3, idx, axis=1).reshape(r * 8, 128)
    s = jnp.sum(d * mex).reshape(1, 1)

    i = pl.program_id(0)

    @pl.when(i == 0)
    def _init():
        o_ref[...] = jnp.zeros((1, 1), jnp.float32)

    o_ref[...] += s


def kernel(uv_points, uv_gt, object_mask):
    B, H, W, C = uv_points.shape
    nrow = B * H * (W // 128) * C
    mrow = B * H * (W // 128)

    def as_rows(a):
        return (
            a.reshape(B, H, W // 128, 128, C)
            .transpose(0, 1, 2, 4, 3)
            .reshape(nrow, 128)
        )

    x = as_rows(uv_points)
    g = as_rows(uv_gt)
    m = object_mask.view(jnp.uint8)

    R = 4096
    n_steps = nrow // R
    out = pl.pallas_call(
        _l1_kernel,
        grid=(n_steps,),
        in_specs=[
            pl.BlockSpec((R, 128), lambda i: (i, 0)),
            pl.BlockSpec((R, 128), lambda i: (i, 0)),
            pl.BlockSpec((R // 4096, 512, 512), lambda i: (i, 0, 0)),
        ],
        out_specs=pl.BlockSpec((1, 1), lambda i: (0, 0)),
        out_shape=jax.ShapeDtypeStruct((1, 1), jnp.float32),
    )(x, g, m)

    uv_loss = out[0, 0] / float(B * H)
    return (_FWD_WEIGHT * uv_loss, uv_loss)


# speedup vs baseline: 1.1111x; 1.1111x over previous
import jax
import jax.numpy as jnp
from jax.experimental import pallas as pl

_FWD_WEIGHT = 1.0


def _l1_kernel(x_ref, g_ref, m_ref, o_ref):
    d = jnp.abs(x_ref[...] - g_ref[...])
    mf = m_ref[...].astype(jnp.float32)
    r = d.shape[0] // 8
    m3 = mf.reshape(r, 4, 128)
    idx = jax.lax.broadcasted_iota(jnp.int32, (r, 8, 128), 1) // 2
    mex = jnp.take_along_axis(m3, idx, axis=1).reshape(r * 8, 128)
    s = jnp.sum(d * mex).reshape(1, 1)

    i = pl.program_id(0)

    @pl.when(i == 0)
    def _init():
        o_ref[...] = jnp.zeros((1, 1), jnp.float32)

    o_ref[...] += s


def kernel(uv_points, uv_gt, object_mask):
    B, H, W, C = uv_points.shape
    nrow = B * H * (W // 128) * C
    mrow = B * H * (W // 128)

    def as_rows(a):
        return (
            a.reshape(B, H, W // 128, 128, C)
            .transpose(0, 1, 2, 4, 3)
            .reshape(nrow, 128)
        )

    x = as_rows(uv_points)
    g = as_rows(uv_gt)
    m = object_mask.view(jnp.uint8)

    R = 8192
    n_steps = nrow // R
    out = pl.pallas_call(
        _l1_kernel,
        grid=(n_steps,),
        in_specs=[
            pl.BlockSpec((R, 128), lambda i: (i, 0)),
            pl.BlockSpec((R, 128), lambda i: (i, 0)),
            pl.BlockSpec((R // 4096, 512, 512), lambda i: (i, 0, 0)),
        ],
        out_specs=pl.BlockSpec((1, 1), lambda i: (0, 0)),
        out_shape=jax.ShapeDtypeStruct((1, 1), jnp.float32),
    )(x, g, m)

    uv_loss = out[0, 0] / float(B * H)
    return (_FWD_WEIGHT * uv_loss, uv_loss)
